# baseline (speedup 1.0000x reference)
import jax
import jax.numpy as jnp
from jax import lax
from jax.experimental import pallas as pl
from jax.experimental.pallas import tpu as pltpu
from jax.experimental.pallas import tpu_sc as plsc

_EMB = 128
_R = 10240
_NC, _NS = 2, 16
_NW = _NC * _NS
_CH = 128
_G = 8
_NCH0 = 80
_NCH1 = 80
_NCHM = max(_NCH0, _NCH1)
_EPAD = _NS * (_NCH0 + _NCH1) * _CH
_DUMMY = 10000
_K = 4

_P0 = 5888
_ACC0 = 6016
_ACC0_TILE = _ACC0 // _NS
_P1R = _R - _P0
_ACC1 = 4480
_ACC1_TILE = _ACC1 // _NS
_CAP1 = (_NCHM + 5) * _CH
_OUT_R = _ACC0 + _ACC1
_SHIFT = 16384


def _sc_degrees(nidx, hidx):
    mesh = plsc.VectorSubcoreMesh(core_axis_name="c", subcore_axis_name="s")
    out_type = [
        jax.ShapeDtypeStruct((_NW, _R), jnp.float32),
        jax.ShapeDtypeStruct((_NW, _R), jnp.float32),
    ]
    scratch = [
        pltpu.VMEM((_NCHM, _CH), jnp.int32),
        pltpu.VMEM((_NCHM, _CH), jnp.int32),
        pltpu.VMEM((_R,), jnp.float32),
        pltpu.VMEM((_R,), jnp.float32),
    ]

    def body(nidx_hbm, hidx_hbm, dn_out, be_out, ni, hi, dnv, bev):
        cid = lax.axis_index("c")
        sid = lax.axis_index("s")
        w = cid * _NS + sid
        ncht = jnp.where(cid == 0, _NCH0, _NCH1)
        pltpu.sync_copy(nidx_hbm.at[cid, sid], ni)
        pltpu.sync_copy(hidx_hbm.at[cid, sid], hi)

        z16 = jnp.zeros((16,), jnp.float32)

        def zero(i, carry):
            dnv[pl.ds(i * 16, 16)] = z16
            bev[pl.ds(i * 16, 16)] = z16
            return carry
        lax.fori_loop(0, _R // 16, zero, 0)

        ones16 = jnp.ones((16,), jnp.float32)

        def hist(i, carry):
            j = i // 8
            v = i % 8
            plsc.addupdate_scatter(dnv, [ni[j, pl.ds(v * 16, 16)]], ones16)
            plsc.addupdate_scatter(bev, [hi[j, pl.ds(v * 16, 16)]], ones16)
            return carry
        lax.fori_loop(0, ncht * 8, hist, 0)

        pltpu.sync_copy(dnv, dn_out.at[w])
        pltpu.sync_copy(bev, be_out.at[w])

    fn = pl.kernel(body, mesh=mesh, out_type=out_type, scratch_types=scratch,
                   compiler_params=pltpu.CompilerParams(
                       needs_layout_passes=False))
    return fn(nidx, hidx)


def _sc_propagate(table, gidx, sidx):
    mesh = plsc.VectorSubcoreMesh(core_axis_name="c", subcore_axis_name="s")
    out_type = [jax.ShapeDtypeStruct((_NC, _OUT_R, _EMB), jnp.float32)]
    scratch = [
        pltpu.VMEM((2, _G, _CH), jnp.int32),
        pltpu.VMEM((2, _G, _CH), jnp.int32),
        pltpu.VMEM((_CAP1,), jnp.int32),
        pltpu.VMEM((8, _CH), jnp.int32),
        pltpu.VMEM((8, _CH), jnp.int32),
        pltpu.VMEM((_CH, _EMB), jnp.float32),
        pltpu.VMEM((_CH, _EMB), jnp.float32),
        pltpu.VMEM((_CH, _EMB), jnp.float32),
        pltpu.VMEM((_CH, _EMB), jnp.float32),
        pltpu.VMEM_SHARED((_ACC0, _EMB), jnp.float32),
        pltpu.SemaphoreType.DMA,
        pltpu.SemaphoreType.DMA,
        pltpu.SemaphoreType.DMA,
        pltpu.SemaphoreType.DMA,
        pltpu.SemaphoreType.DMA,
        pltpu.SemaphoreType.DMA,
        pltpu.SemaphoreType.DMA,
        pltpu.SemaphoreType.DMA,
        pltpu.SemaphoreType.DMA,
    ]

    def body(table_hbm, gidx_hbm, sidx_hbm, acc_out,
             gib, sgb, cw, grow, csrow, buf0, buf1, buf2, buf3, acc,
             gs0, gs1, gs2, gs3, ss0, ss1, ss2, ss3, semc):
        bufs = (buf0, buf1, buf2, buf3)
        gsem = (gs0, gs1, gs2, gs3)
        ssem = (ss0, ss1, ss2, ss3)
        cid = lax.axis_index("c")
        sid = lax.axis_index("s")
        w = cid * _NS + sid

        z16 = jnp.zeros((16,), jnp.float32)

        def zero_buf(r, carry):
            for q in range(_EMB // 16):
                buf3[r, pl.ds(q * 16, 16)] = z16
            return carry
        lax.fori_loop(0, _CH, zero_buf, 0)

        def zero_range(base, n):
            cps = []
            done = 0
            while done < n:
                m = min(_CH, n - done)
                cps.append(pltpu.async_copy(
                    buf3.at[pl.ds(0, m)],
                    acc.at[pl.ds(base + done, m)], semc))
                done += m
            for c in cps:
                c.wait()

        base0 = sid * _ACC0_TILE
        zero_range(base0, _ACC0_TILE)

        ngt = jnp.where(cid == 0, _NCH0 // _G, _NCH1 // _G)
        pltpu.sync_copy(gidx_hbm.at[cid, sid, pl.ds(0, _G)], gib.at[0])
        pltpu.sync_copy(sidx_hbm.at[cid, sid, pl.ds(0, _G)], sgb.at[0])

        plsc.subcore_barrier()

        p0 = jnp.full((16,), _P0, jnp.int32)

        def valu(slot, j, off):
            for v in range(_CH // 16):
                sv = sgb[slot, j, pl.ds(v * 16, 16)]
                gv = gib[slot, j, pl.ds(v * 16, 16)]
                sgb[slot, j, pl.ds(v * 16, 16)] = jnp.minimum(sv, p0)
                m = sv >= p0
                pk = gv + (sv - p0) * _SHIFT
                plsc.store_compressed(cw.at[pl.ds(off, 16)], pk, mask=m)
                off = off + jnp.sum(m.astype(jnp.int32))
            return off

        for b in range(_K):
            pltpu.async_copy(table_hbm.at[gib.at[0, b]], bufs[b], gsem[b])

        def group(g, off):
            cur = g % 2
            nxt = (g + 1) % 2
            gnext = jnp.minimum(g + 1, ngt - 1)
            pg = pltpu.async_copy(gidx_hbm.at[cid, sid, pl.ds(gnext * _G, _G)],
                                  gib.at[nxt], semc)
            ps = pltpu.async_copy(sidx_hbm.at[cid, sid, pl.ds(gnext * _G, _G)],
                                  sgb.at[nxt], semc)
            scat = [None] * _K
            for sub in range(_G // _K):
                gats = [None] * _K
                if sub > 0:
                    for b in range(_K):
                        scat[b].wait()
                        gats[b] = pltpu.async_copy(
                            table_hbm.at[gib.at[cur, sub * _K + b]],
                            bufs[b], gsem[b])
                for b in range(_K):
                    j = sub * _K + b
                    if sub == 0:
                        pltpu.make_async_copy(
                            table_hbm.at[gib.at[cur, b]], bufs[b],
                            gsem[b]).wait()
                    else:
                        gats[b].wait()
                    off = valu(cur, j, off)
                    scat[b] = pltpu.async_copy(
                        bufs[b], acc.at[sgb.at[cur, j]], ssem[b], add=True)
            pg.wait()
            ps.wait()
            for b in range(_K):
                scat[b].wait()
                pltpu.async_copy(table_hbm.at[gib.at[nxt, b]],
                                 bufs[b], gsem[b])
            return off
        off = lax.fori_loop(0, ngt, group, jnp.int32(0))

        for b in range(_K):
            pltpu.make_async_copy(
                table_hbm.at[gib.at[0, b]], bufs[b], gsem[b]).wait()

        plsc.subcore_barrier()
        pltpu.sync_copy(acc.at[pl.ds(base0, _ACC0_TILE)],
                        acc_out.at[cid, pl.ds(base0, _ACC0_TILE)])
        plsc.subcore_barrier()

        pad16 = jnp.full((16,), _P1R * _SHIFT, jnp.int32)

        def padw(k, carry):
            cw[pl.ds(off + k * 16, 16)] = pad16
            return carry
        lax.fori_loop(0, (_K * _CH + 16) // 16, padw, 0)
        nquad = (off + _K * _CH - 1) // (_K * _CH)

        lax.fori_loop(0, _CH, zero_buf, 0)
        base1 = sid * _ACC1_TILE
        zero_range(base1, _ACC1_TILE)

        plsc.subcore_barrier()

        mask14 = jnp.full((16,), _SHIFT - 1, jnp.int32)

        def quad(t, carry):
            gats = [None] * _K
            for b in range(_K):
                @pl.when(t > 0)
                def _wait_prev():
                    pltpu.make_async_copy(bufs[b], acc.at[csrow.at[b]],
                                          ssem[b]).wait()
                jc = (t * _K + b) * _CH
                for v in range(_CH // 16):
                    pk = cw[pl.ds(jc + v * 16, 16)]
                    grow[b, pl.ds(v * 16, 16)] = pk & mask14
                    csrow[b, pl.ds(v * 16, 16)] = pk // _SHIFT
                gats[b] = pltpu.async_copy(table_hbm.at[grow.at[b]],
                                           bufs[b], gsem[b])
            for b in range(_K):
                gats[b].wait()
                pltpu.async_copy(bufs[b], acc.at[csrow.at[b]],
                                 ssem[b], add=True)
            return carry
        lax.fori_loop(0, nquad, quad, 0)

        @pl.when(nquad > 0)
        def _drain_ph1():
            for b in range(_K):
                pltpu.make_async_copy(bufs[b], acc.at[csrow.at[b]],
                                      ssem[b]).wait()

        plsc.subcore_barrier()
        pltpu.sync_copy(acc.at[pl.ds(base1, _ACC1_TILE)],
                        acc_out.at[cid, pl.ds(_ACC0 + base1, _ACC1_TILE)])

    fn = pl.kernel(body, mesh=mesh, out_type=out_type, scratch_types=scratch,
                   compiler_params=pltpu.CompilerParams(
                       needs_layout_passes=False))
    outs = fn(table, gidx, sidx)
    return outs[0] if isinstance(outs, (list, tuple)) else outs


def _tc_combine_scale(parts, deg_parts):
    def body(p_ref, d_ref, o_ref):
        deg = jnp.sum(d_ref[...], axis=0)
        inv = jnp.where(deg > 0, 1.0 / deg, 0.0)
        comb0 = p_ref[0, :_P0] + p_ref[1, :_P0]
        comb1 = (p_ref[0, _ACC0:_ACC0 + _P1R] + p_ref[1, _ACC0:_ACC0 + _P1R])
        o_ref[:_P0] = comb0 * inv[:_P0, None]
        o_ref[_P0:] = comb1 * inv[_P0:, None]

    return pl.pallas_call(
        body,
        out_shape=jax.ShapeDtypeStruct((_R, _EMB), jnp.float32),
    )(parts, deg_parts)


def _tc_finalize(z_parts, dn_parts, w0, b0, w1, b1):
    def body(zp, dp, w0r, b0r, w1r, b1r, o_ref):
        deg = jnp.sum(dp[...], axis=0)
        inv = jnp.where(deg > 0, 1.0 / deg, 0.0)
        z0 = (zp[0, :_P0] + zp[1, :_P0]) * inv[:_P0, None]
        z1 = ((zp[0, _ACC0:_ACC0 + _P1R] + zp[1, _ACC0:_ACC0 + _P1R])
              * inv[_P0:, None])
        z = jnp.concatenate([z0, z1], axis=0)
        h0 = jnp.dot(z, w0r[...], preferred_element_type=jnp.float32) + b0r[...]
        h0 = h0 / jnp.maximum(
            jnp.sqrt(jnp.sum(h0 * h0, axis=-1, keepdims=True)), 1e-12)
        h1 = jnp.dot(z, w1r[...], preferred_element_type=jnp.float32) + b1r[...]
        h1 = h1 / jnp.maximum(
            jnp.sqrt(jnp.sum(h1 * h1, axis=-1, keepdims=True)), 1e-12)
        o_ref[...] = h0 + h1

    return pl.pallas_call(
        body,
        out_shape=jax.ShapeDtypeStruct((_R, _EMB), jnp.float32),
    )(z_parts, dn_parts, w0, b0, w1, b1)


def _split_cores(idx):
    ec = idx.reshape(-1, _CH)
    c0 = ec[:_NS * _NCH0].reshape(_NS, _NCH0, _CH)
    c1 = ec[_NS * _NCH0:].reshape(_NS, _NCH1, _CH)
    c0 = jnp.pad(c0, ((0, 0), (0, _NCHM - _NCH0), (0, 0)),
                 constant_values=_DUMMY)
    c1 = jnp.pad(c1, ((0, 0), (0, _NCHM - _NCH1), (0, 0)),
                 constant_values=_DUMMY)
    return jnp.stack([c0, c1])


def kernel(x, edge_index, W0, b0, W1, b1):
    node_idx = edge_index[0]
    hedge_idx = edge_index[1]
    n = x.shape[0]
    e = node_idx.shape[0]
    padlen = _EPAD - e
    pad = jnp.full((padlen,), _DUMMY, jnp.int32)
    nidx = _split_cores(jnp.concatenate([node_idx, pad]))
    hidx = _split_cores(jnp.concatenate([hedge_idx, pad]))
    xp = jnp.zeros((_R, _EMB), jnp.float32).at[:n].set(x)

    dn_parts, be_parts = _sc_degrees(nidx, hidx)
    he_parts = _sc_propagate(xp, nidx, hidx)
    he = _tc_combine_scale(he_parts, be_parts)
    z_parts = _sc_propagate(he, hidx, nidx)
    out = _tc_finalize(z_parts, dn_parts,
                       W0, b0.reshape(1, _EMB), W1, b1.reshape(1, _EMB))
    return out[:n]


# speedup vs baseline: 1.1996x; 1.1996x over previous
import jax
import jax.numpy as jnp
from jax import lax
from jax.experimental import pallas as pl
from jax.experimental.pallas import tpu as pltpu
from jax.experimental.pallas import tpu_sc as plsc

_EMB = 128
_R = 10240
_NC, _NS = 2, 16
_NW = _NC * _NS
_CH = 128
_G = 8
_NCH0 = 48
_NCH1 = 112
_NCHM = max(_NCH0, _NCH1)
_EPAD = _NS * (_NCH0 + _NCH1) * _CH
_DUMMY = 10000
_K = 4

_P0 = 5376
_ACC0 = 5504
_ACC0_TILE = _ACC0 // _NS
_P1R = _R - _P0
_ACC1 = 4992
_ACC1_TILE = _ACC1 // _NS
_CAP1 = (_NCHM + 5) * _CH
_OUT_R = _ACC0 + _ACC1
_SHIFT = 16384


def _sc_degrees(nidx, hidx):
    mesh = plsc.VectorSubcoreMesh(core_axis_name="c", subcore_axis_name="s")
    out_type = [
        jax.ShapeDtypeStruct((_NW, _R), jnp.float32),
        jax.ShapeDtypeStruct((_NW, _R), jnp.float32),
    ]
    scratch = [
        pltpu.VMEM((_NCHM, _CH), jnp.int32),
        pltpu.VMEM((_NCHM, _CH), jnp.int32),
        pltpu.VMEM((_R,), jnp.float32),
        pltpu.VMEM((_R,), jnp.float32),
    ]

    def body(nidx_hbm, hidx_hbm, dn_out, be_out, ni, hi, dnv, bev):
        cid = lax.axis_index("c")
        sid = lax.axis_index("s")
        w = cid * _NS + sid
        ncht = jnp.where(cid == 0, _NCH0, _NCH1)
        pltpu.sync_copy(nidx_hbm.at[cid, sid], ni)
        pltpu.sync_copy(hidx_hbm.at[cid, sid], hi)

        z16 = jnp.zeros((16,), jnp.float32)

        def zero(i, carry):
            dnv[pl.ds(i * 16, 16)] = z16
            bev[pl.ds(i * 16, 16)] = z16
            return carry
        lax.fori_loop(0, _R // 16, zero, 0)

        ones16 = jnp.ones((16,), jnp.float32)

        def hist(i, carry):
            j = i // 8
            v = i % 8
            plsc.addupdate_scatter(dnv, [ni[j, pl.ds(v * 16, 16)]], ones16)
            plsc.addupdate_scatter(bev, [hi[j, pl.ds(v * 16, 16)]], ones16)
            return carry
        lax.fori_loop(0, ncht * 8, hist, 0)

        pltpu.sync_copy(dnv, dn_out.at[w])
        pltpu.sync_copy(bev, be_out.at[w])

    fn = pl.kernel(body, mesh=mesh, out_type=out_type, scratch_types=scratch,
                   compiler_params=pltpu.CompilerParams(
                       needs_layout_passes=False))
    return fn(nidx, hidx)


def _sc_propagate(table, gidx, sidx):
    mesh = plsc.VectorSubcoreMesh(core_axis_name="c", subcore_axis_name="s")
    out_type = [jax.ShapeDtypeStruct((_NC, _OUT_R, _EMB), jnp.float32)]
    scratch = [
        pltpu.VMEM((2, _G, _CH), jnp.int32),
        pltpu.VMEM((2, _G, _CH), jnp.int32),
        pltpu.VMEM((_CAP1,), jnp.int32),
        pltpu.VMEM((8, _CH), jnp.int32),
        pltpu.VMEM((8, _CH), jnp.int32),
        pltpu.VMEM((_CH, _EMB), jnp.float32),
        pltpu.VMEM((_CH, _EMB), jnp.float32),
        pltpu.VMEM((_CH, _EMB), jnp.float32),
        pltpu.VMEM((_CH, _EMB), jnp.float32),
        pltpu.VMEM_SHARED((_ACC0, _EMB), jnp.float32),
        pltpu.SemaphoreType.DMA,
        pltpu.SemaphoreType.DMA,
        pltpu.SemaphoreType.DMA,
        pltpu.SemaphoreType.DMA,
        pltpu.SemaphoreType.DMA,
        pltpu.SemaphoreType.DMA,
        pltpu.SemaphoreType.DMA,
        pltpu.SemaphoreType.DMA,
        pltpu.SemaphoreType.DMA,
    ]

    def body(table_hbm, gidx_hbm, sidx_hbm, acc_out,
             gib, sgb, cw, grow, csrow, buf0, buf1, buf2, buf3, acc,
             gs0, gs1, gs2, gs3, ss0, ss1, ss2, ss3, semc):
        bufs = (buf0, buf1, buf2, buf3)
        gsem = (gs0, gs1, gs2, gs3)
        ssem = (ss0, ss1, ss2, ss3)
        cid = lax.axis_index("c")
        sid = lax.axis_index("s")
        w = cid * _NS + sid

        z16 = jnp.zeros((16,), jnp.float32)

        def zero_buf(r, carry):
            for q in range(_EMB // 16):
                buf3[r, pl.ds(q * 16, 16)] = z16
            return carry
        lax.fori_loop(0, _CH, zero_buf, 0)

        def zero_range(base, n):
            cps = []
            done = 0
            while done < n:
                m = min(_CH, n - done)
                cps.append(pltpu.async_copy(
                    buf3.at[pl.ds(0, m)],
                    acc.at[pl.ds(base + done, m)], semc))
                done += m
            for c in cps:
                c.wait()

        base0 = sid * _ACC0_TILE
        zero_range(base0, _ACC0_TILE)

        ngt = jnp.where(cid == 0, _NCH0 // _G, _NCH1 // _G)
        pltpu.sync_copy(gidx_hbm.at[cid, sid, pl.ds(0, _G)], gib.at[0])
        pltpu.sync_copy(sidx_hbm.at[cid, sid, pl.ds(0, _G)], sgb.at[0])

        plsc.subcore_barrier()

        p0 = jnp.full((16,), _P0, jnp.int32)

        def valu(slot, j, off):
            for v in range(_CH // 16):
                sv = sgb[slot, j, pl.ds(v * 16, 16)]
                gv = gib[slot, j, pl.ds(v * 16, 16)]
                sgb[slot, j, pl.ds(v * 16, 16)] = jnp.minimum(sv, p0)
                m = sv >= p0
                pk = gv + (sv - p0) * _SHIFT
                plsc.store_compressed(cw.at[pl.ds(off, 16)], pk, mask=m)
                off = off + jnp.sum(m.astype(jnp.int32))
            return off

        for b in range(_K):
            pltpu.async_copy(table_hbm.at[gib.at[0, b]], bufs[b], gsem[b])

        def group(g, off):
            cur = g % 2
            nxt = (g + 1) % 2
            gnext = jnp.minimum(g + 1, ngt - 1)
            pg = pltpu.async_copy(gidx_hbm.at[cid, sid, pl.ds(gnext * _G, _G)],
                                  gib.at[nxt], semc)
            ps = pltpu.async_copy(sidx_hbm.at[cid, sid, pl.ds(gnext * _G, _G)],
                                  sgb.at[nxt], semc)
            scat = [None] * _K
            for sub in range(_G // _K):
                gats = [None] * _K
                if sub > 0:
                    for b in range(_K):
                        scat[b].wait()
                        gats[b] = pltpu.async_copy(
                            table_hbm.at[gib.at[cur, sub * _K + b]],
                            bufs[b], gsem[b])
                for b in range(_K):
                    j = sub * _K + b
                    if sub == 0:
                        pltpu.make_async_copy(
                            table_hbm.at[gib.at[cur, b]], bufs[b],
                            gsem[b]).wait()
                    else:
                        gats[b].wait()
                    off = valu(cur, j, off)
                    scat[b] = pltpu.async_copy(
                        bufs[b], acc.at[sgb.at[cur, j]], ssem[b], add=True)
            pg.wait()
            ps.wait()
            for b in range(_K):
                scat[b].wait()
                pltpu.async_copy(table_hbm.at[gib.at[nxt, b]],
                                 bufs[b], gsem[b])
            return off
        off = lax.fori_loop(0, ngt, group, jnp.int32(0))

        for b in range(_K):
            pltpu.make_async_copy(
                table_hbm.at[gib.at[0, b]], bufs[b], gsem[b]).wait()

        plsc.subcore_barrier()
        pltpu.sync_copy(acc.at[pl.ds(base0, _ACC0_TILE)],
                        acc_out.at[cid, pl.ds(base0, _ACC0_TILE)])
        plsc.subcore_barrier()

        pad16 = jnp.full((16,), _P1R * _SHIFT, jnp.int32)

        def padw(k, carry):
            cw[pl.ds(off + k * 16, 16)] = pad16
            return carry
        lax.fori_loop(0, (_K * _CH + 16) // 16, padw, 0)
        nquad = (off + _K * _CH - 1) // (_K * _CH)

        lax.fori_loop(0, _CH, zero_buf, 0)
        base1 = sid * _ACC1_TILE
        zero_range(base1, _ACC1_TILE)

        plsc.subcore_barrier()

        mask14 = jnp.full((16,), _SHIFT - 1, jnp.int32)

        def quad(t, carry):
            gats = [None] * _K
            for b in range(_K):
                jc = (t * _K + b) * _CH
                for v in range(_CH // 16):
                    pk = cw[pl.ds(jc + v * 16, 16)]
                    grow[b, pl.ds(v * 16, 16)] = pk & mask14
                    csrow[b, pl.ds(v * 16, 16)] = pk // _SHIFT
                gats[b] = pltpu.async_copy(table_hbm.at[grow.at[b]],
                                           bufs[b], gsem[b])
            scat = [None] * _K
            for b in range(_K):
                gats[b].wait()
                scat[b] = pltpu.async_copy(bufs[b], acc.at[csrow.at[b]],
                                           ssem[b], add=True)
            for b in range(_K):
                scat[b].wait()
            return carry
        lax.fori_loop(0, nquad, quad, 0)

        plsc.subcore_barrier()
        pltpu.sync_copy(acc.at[pl.ds(base1, _ACC1_TILE)],
                        acc_out.at[cid, pl.ds(_ACC0 + base1, _ACC1_TILE)])

    fn = pl.kernel(body, mesh=mesh, out_type=out_type, scratch_types=scratch,
                   compiler_params=pltpu.CompilerParams(
                       needs_layout_passes=False))
    outs = fn(table, gidx, sidx)
    return outs[0] if isinstance(outs, (list, tuple)) else outs


def _tc_combine_scale(parts, deg_parts):
    def body(p_ref, d_ref, o_ref):
        deg = jnp.sum(d_ref[...], axis=0)
        inv = jnp.where(deg > 0, 1.0 / deg, 0.0)
        comb0 = p_ref[0, :_P0] + p_ref[1, :_P0]
        comb1 = (p_ref[0, _ACC0:_ACC0 + _P1R] + p_ref[1, _ACC0:_ACC0 + _P1R])
        o_ref[:_P0] = comb0 * inv[:_P0, None]
        o_ref[_P0:] = comb1 * inv[_P0:, None]

    return pl.pallas_call(
        body,
        out_shape=jax.ShapeDtypeStruct((_R, _EMB), jnp.float32),
    )(parts, deg_parts)


def _tc_finalize(z_parts, dn_parts, w0, b0, w1, b1):
    def body(zp, dp, w0r, b0r, w1r, b1r, o_ref):
        deg = jnp.sum(dp[...], axis=0)
        inv = jnp.where(deg > 0, 1.0 / deg, 0.0)
        z0 = (zp[0, :_P0] + zp[1, :_P0]) * inv[:_P0, None]
        z1 = ((zp[0, _ACC0:_ACC0 + _P1R] + zp[1, _ACC0:_ACC0 + _P1R])
              * inv[_P0:, None])
        z = jnp.concatenate([z0, z1], axis=0)
        h0 = jnp.dot(z, w0r[...], preferred_element_type=jnp.float32) + b0r[...]
        h0 = h0 / jnp.maximum(
            jnp.sqrt(jnp.sum(h0 * h0, axis=-1, keepdims=True)), 1e-12)
        h1 = jnp.dot(z, w1r[...], preferred_element_type=jnp.float32) + b1r[...]
        h1 = h1 / jnp.maximum(
            jnp.sqrt(jnp.sum(h1 * h1, axis=-1, keepdims=True)), 1e-12)
        o_ref[...] = h0 + h1

    return pl.pallas_call(
        body,
        out_shape=jax.ShapeDtypeStruct((_R, _EMB), jnp.float32),
    )(z_parts, dn_parts, w0, b0, w1, b1)


def _split_cores(idx):
    ec = idx.reshape(-1, _CH)
    c0 = ec[:_NS * _NCH0].reshape(_NS, _NCH0, _CH)
    c1 = ec[_NS * _NCH0:].reshape(_NS, _NCH1, _CH)
    c0 = jnp.pad(c0, ((0, 0), (0, _NCHM - _NCH0), (0, 0)),
                 constant_values=_DUMMY)
    c1 = jnp.pad(c1, ((0, 0), (0, _NCHM - _NCH1), (0, 0)),
                 constant_values=_DUMMY)
    return jnp.stack([c0, c1])


def kernel(x, edge_index, W0, b0, W1, b1):
    node_idx = edge_index[0]
    hedge_idx = edge_index[1]
    n = x.shape[0]
    e = node_idx.shape[0]
    padlen = _EPAD - e
    pad = jnp.full((padlen,), _DUMMY, jnp.int32)
    nidx = _split_cores(jnp.concatenate([node_idx, pad]))
    hidx = _split_cores(jnp.concatenate([hedge_idx, pad]))
    xp = jnp.zeros((_R, _EMB), jnp.float32).at[:n].set(x)

    dn_parts, be_parts = _sc_degrees(nidx, hidx)
    he_parts = _sc_propagate(xp, nidx, hidx)
    he = _tc_combine_scale(he_parts, be_parts)
    z_parts = _sc_propagate(he, hidx, nidx)
    out = _tc_finalize(z_parts, dn_parts,
                       W0, b0.reshape(1, _EMB), W1, b1.reshape(1, _EMB))
    return out[:n]
